# Initial kernel scaffold; baseline (speedup 1.0000x reference)
import jax
import jax.numpy as jnp
from jax.experimental import pallas as pl


def kernel(x, edge_index, W_l, b_l, W_r, gamma, beta):
    raise NotImplementedError("write your pallas kernel here")


# speedup vs baseline: 2.8156x; 2.8156x over previous
import jax
import jax.numpy as jnp
from jax import lax
from jax.experimental import pallas as pl
from jax.experimental.pallas import tpu as pltpu
from jax.experimental.pallas import tpu_sc as plsc

N = 10000
E = 320000
D = 128
EPS = 1e-5

NSC = 2
NTEC = 16
CH = 128
CHUNKS_PER_TILE = -(-E // (CH * NTEC))
E_PAD = CHUNKS_PER_TILE * CH * NTEC
NHALF = 5120
NHROWS = NHALF + 8


def _dloc(dst_v, dloc_v, lo):
    for g in range(CH // 16):
        dv = dst_v[pl.ds(g * 16, 16)]
        inr = jnp.logical_and(dv >= lo, dv < lo + NHALF)
        dloc_v[pl.ds(g * 16, 16)] = jnp.where(inr, dv - lo, NHALF)


def _sc_agg(x, src, dst, zrow):
    mesh = plsc.VectorSubcoreMesh(core_axis_name="c", subcore_axis_name="s")

    def body(x_hbm, src_hbm, dst_hbm, zrow_hbm, agg_out,
             src_v, dst_v, dloc_v, rows_v, agg_sh, sem):
        c = lax.axis_index("c")
        s = lax.axis_index("s")
        lo = c * NHALF

        @pl.when(s == 0)
        def _():
            pltpu.sync_copy(zrow_hbm, agg_sh)

        plsc.subcore_barrier()

        def chunk(cix, _):
            off = (s * CHUNKS_PER_TILE + cix) * CH
            pltpu.sync_copy(src_hbm.at[pl.ds(off, CH)], src_v)
            pltpu.sync_copy(dst_hbm.at[pl.ds(off, CH)], dst_v)
            pltpu.async_copy(x_hbm.at[src_v], rows_v, sem).wait()
            _dloc(dst_v, dloc_v, lo)
            pltpu.sync_copy(rows_v, agg_sh.at[dloc_v], add=True)
            return 0
        lax.fori_loop(0, CHUNKS_PER_TILE, chunk, 0)

        plsc.subcore_barrier()

        @pl.when(s == 0)
        def _():
            pltpu.sync_copy(agg_sh, agg_out.at[c])

    f = pl.kernel(
        body,
        out_type=jax.ShapeDtypeStruct((NSC, NHROWS, D), jnp.float32),
        mesh=mesh,
        scratch_types=(
            pltpu.VMEM((CH,), jnp.int32),
            pltpu.VMEM((CH,), jnp.int32),
            pltpu.VMEM((CH,), jnp.int32),
            pltpu.VMEM((CH, D), jnp.float32),
            pltpu.VMEM_SHARED((NHROWS, D), jnp.float32),
            pltpu.SemaphoreType.DMA,
        ),
    )
    return f(x, src, dst, zrow)


def _sc_cnt(dst, zrow):
    mesh = plsc.VectorSubcoreMesh(core_axis_name="c", subcore_axis_name="s")

    def body(dst_hbm, zrow_hbm, cnt_out, dst_v, dloc_v, ones_v, cnt_sh):
        c = lax.axis_index("c")
        s = lax.axis_index("s")
        lo = c * NHALF

        one16 = jnp.ones((16,), jnp.float32)

        def ofill(i, _):
            for j in range(D // 16):
                ones_v[i, pl.ds(j * 16, 16)] = one16
            return 0
        lax.fori_loop(0, CH, ofill, 0)

        @pl.when(s == 0)
        def _():
            pltpu.sync_copy(zrow_hbm, cnt_sh)

        plsc.subcore_barrier()

        def chunk(cix, _):
            off = (s * CHUNKS_PER_TILE + cix) * CH
            pltpu.sync_copy(dst_hbm.at[pl.ds(off, CH)], dst_v)
            _dloc(dst_v, dloc_v, lo)
            pltpu.sync_copy(ones_v, cnt_sh.at[dloc_v], add=True)
            return 0
        lax.fori_loop(0, CHUNKS_PER_TILE, chunk, 0)

        plsc.subcore_barrier()

        @pl.when(s == 0)
        def _():
            pltpu.sync_copy(cnt_sh, cnt_out.at[c])

    f = pl.kernel(
        body,
        out_type=jax.ShapeDtypeStruct((NSC, NHROWS, D), jnp.float32),
        mesh=mesh,
        scratch_types=(
            pltpu.VMEM((CH,), jnp.int32),
            pltpu.VMEM((CH,), jnp.int32),
            pltpu.VMEM((CH, D), jnp.float32),
            pltpu.VMEM_SHARED((NHROWS, D), jnp.float32),
        ),
    )
    return f(dst, zrow)


BLK = 2000
NBLK = N // BLK


def _tc_dense(agg, cnt, x, W_l, b_l, W_r, gamma, beta):
    def body_a(agg_ref, cnt_ref, x_ref, wl_ref, bl_ref, wr_ref,
               xraw_ref, stats_ref):
        @pl.when(pl.program_id(0) == 0)
        def _():
            stats_ref[...] = jnp.zeros_like(stats_ref)

        mean = agg_ref[...] / jnp.maximum(cnt_ref[...], 1.0)
        xr = lax.dot_general(mean, wl_ref[...], (((1,), (1,)), ((), ())),
                             preferred_element_type=jnp.float32,
                             precision=lax.Precision.HIGHEST)
        xr = xr + bl_ref[...]
        xr = xr + lax.dot_general(x_ref[...], wr_ref[...], (((1,), (1,)), ((), ())),
                                  preferred_element_type=jnp.float32,
                                  precision=lax.Precision.HIGHEST)
        xraw_ref[...] = xr
        stats_ref[0:1, :] += jnp.sum(xr, axis=0, keepdims=True)
        stats_ref[1:2, :] += jnp.sum(xr * xr, axis=0, keepdims=True)

    x_raw, stats = pl.pallas_call(
        body_a,
        grid=(NBLK,),
        in_specs=[
            pl.BlockSpec((BLK, D), lambda i: (i, 0)),
            pl.BlockSpec((BLK, 1), lambda i: (i, 0)),
            pl.BlockSpec((BLK, D), lambda i: (i, 0)),
            pl.BlockSpec((D, D), lambda i: (0, 0)),
            pl.BlockSpec((1, D), lambda i: (0, 0)),
            pl.BlockSpec((D, D), lambda i: (0, 0)),
        ],
        out_specs=(
            pl.BlockSpec((BLK, D), lambda i: (i, 0)),
            pl.BlockSpec((8, D), lambda i: (0, 0)),
        ),
        out_shape=(
            jax.ShapeDtypeStruct((N, D), jnp.float32),
            jax.ShapeDtypeStruct((8, D), jnp.float32),
        ),
    )(agg, cnt, x, W_l, b_l, W_r)

    def body_b(xr_ref, stats_ref, g_ref, b_ref, xd_ref):
        mu = stats_ref[0:1, :] * (1.0 / N)
        var = stats_ref[1:2, :] * (1.0 / N) - mu * mu
        xr = xr_ref[...]
        xd_ref[...] = g_ref[...] * (xr - mu) * lax.rsqrt(var + EPS) + b_ref[...]

    x_deskewed = pl.pallas_call(
        body_b,
        grid=(NBLK,),
        in_specs=[
            pl.BlockSpec((BLK, D), lambda i: (i, 0)),
            pl.BlockSpec((8, D), lambda i: (0, 0)),
            pl.BlockSpec((1, D), lambda i: (0, 0)),
            pl.BlockSpec((1, D), lambda i: (0, 0)),
        ],
        out_specs=pl.BlockSpec((BLK, D), lambda i: (i, 0)),
        out_shape=jax.ShapeDtypeStruct((N, D), jnp.float32),
    )(x_raw, stats, gamma, beta)
    return x_raw, x_deskewed


def kernel(x, edge_index, W_l, b_l, W_r, gamma, beta):
    src = edge_index[0]
    dst = edge_index[1]
    npad = E_PAD - E
    src_p = jnp.concatenate([src, jnp.zeros((npad,), jnp.int32)])
    dst_p = jnp.concatenate([dst, jnp.full((npad,), N, jnp.int32)])
    zrow = jnp.zeros((NHROWS, D), jnp.float32)
    aggp = _sc_agg(x, src_p, dst_p, zrow)
    cntp = _sc_cnt(dst_p, zrow)
    agg = aggp[:, :NHALF].reshape(2 * NHALF, D)[:N]
    cnt = cntp[:, :NHALF, 0:1].reshape(2 * NHALF, 1)[:N]
    x_raw, x_deskewed = _tc_dense(
        agg, cnt, x, W_l,
        b_l.reshape(1, D), W_r, gamma.reshape(1, D), beta.reshape(1, D))
    return (x_raw, x_deskewed)
